# Initial kernel scaffold; baseline (speedup 1.0000x reference)
import jax
import jax.numpy as jnp
from jax.experimental import pallas as pl


def kernel(x, seq_len, edge_index, edge_attr, pos, params):
    raise NotImplementedError("write your pallas kernel here")


# speedup vs baseline: 10.4032x; 10.4032x over previous
import functools

import jax
import jax.numpy as jnp
import numpy as np
from jax import lax
from jax.experimental import pallas as pl
from jax.experimental.pallas import tpu as pltpu
from jax.experimental.pallas import tpu_sc as plsc

N = 10000
E = 320000
K = 256
H = 128
NPAD = 10240
FOLD = NPAD // K
COUT = 3
NMP = 2

NW = 32
EPW = E // NW
ROWS_B = NPAD // NW
SUB_ROWS = NPAD // 16


def _sc_build_b(src, dst, ew):
    mesh = plsc.VectorSubcoreMesh(core_axis_name="c", subcore_axis_name="s")
    CBLK = 2000
    NBLK = E // CBLK

    @functools.partial(
        pl.kernel,
        out_type=jax.ShapeDtypeStruct((NPAD * K,), jnp.float32),
        mesh=mesh,
        compiler_params=pltpu.CompilerParams(needs_layout_passes=False),
        scratch_types=[
            pltpu.VMEM((ROWS_B * K,), jnp.float32),
            pltpu.VMEM((CBLK,), jnp.int32),
            pltpu.VMEM((CBLK,), jnp.int32),
            pltpu.VMEM((CBLK,), jnp.float32),
        ],
    )
    def build(src_hbm, dst_hbm, ew_hbm, b_hbm, bt, sb, db, wb):
        wid = lax.axis_index("s") * 2 + lax.axis_index("c")
        base = wid * ROWS_B
        zero = jnp.zeros((16,), jnp.float32)

        def zrow(i, _):
            bt[pl.ds(i * 16, 16)] = zero
            return 0
        lax.fori_loop(0, ROWS_B * K // 16, zrow, 0)

        def blk(b, _):
            e0 = b * CBLK
            pltpu.sync_copy(src_hbm.at[pl.ds(e0, CBLK)], sb)
            pltpu.sync_copy(dst_hbm.at[pl.ds(e0, CBLK)], db)
            pltpu.sync_copy(ew_hbm.at[pl.ds(e0, CBLK)], wb)

            def grp(g, _):
                d = db[pl.ds(g * 16, 16)]
                s = sb[pl.ds(g * 16, 16)]
                w = wb[pl.ds(g * 16, 16)]
                hs = lax.bitwise_and(s, K - 1)
                r = d - base
                m = (d >= base) & (d < base + ROWS_B)
                idx = jnp.where(m, lax.shift_left(r, 8) + hs, 0)
                plsc.addupdate_scatter(bt, [idx], w, mask=m)
                return 0
            lax.fori_loop(0, CBLK // 16, grp, 0)
            return 0
        lax.fori_loop(0, NBLK, blk, 0)
        pltpu.sync_copy(bt, b_hbm.at[pl.ds(base * K, ROWS_B * K)])

    return build(src, dst, ew)


def _sc_propagate(table, src, dst, ew):
    mesh = plsc.VectorSubcoreMesh(core_axis_name="c", subcore_axis_name="s")
    CB = 2000
    CG = 80
    NBLK = EPW // CB
    NCH = CB // CG

    @functools.partial(
        pl.kernel,
        out_type=jax.ShapeDtypeStruct((2, NPAD, H), jnp.float32),
        mesh=mesh,
        compiler_params=pltpu.CompilerParams(needs_layout_passes=False),
        scratch_types=[
            pltpu.VMEM_SHARED((NPAD, H), jnp.float32),
            pltpu.VMEM((CG, H), jnp.float32),
            pltpu.VMEM((80, H), jnp.float32),
            pltpu.VMEM((CB,), jnp.int32),
            pltpu.VMEM((CB,), jnp.int32),
            pltpu.VMEM((CB,), jnp.float32),
            pltpu.VMEM((CG,), jnp.int32),
            pltpu.VMEM((CG,), jnp.int32),
            pltpu.SemaphoreType.DMA,
        ],
    )
    def prop(tab_hbm, src_hbm, dst_hbm, ew_hbm, out_hbm,
             acc, rows, zbuf, sb, db, wb, sc, dc, sem):
        cid = lax.axis_index("c")
        sid = lax.axis_index("s")
        wid = sid * 2 + cid
        zero = jnp.zeros((16,), jnp.float32)

        def zrow(i, _):
            for j in range(H // 16):
                zbuf[i, pl.ds(j * 16, 16)] = zero
            return 0
        lax.fori_loop(0, 80, zrow, 0)

        def zstripe(i, _):
            pltpu.sync_copy(zbuf, acc.at[pl.ds(sid * SUB_ROWS + i * 80, 80)])
            return 0
        lax.fori_loop(0, SUB_ROWS // 80, zstripe, 0)
        plsc.subcore_barrier()

        def blk(b, _):
            e0 = wid * EPW + b * CB
            pltpu.sync_copy(src_hbm.at[pl.ds(e0, CB)], sb)
            pltpu.sync_copy(dst_hbm.at[pl.ds(e0, CB)], db)
            pltpu.sync_copy(ew_hbm.at[pl.ds(e0, CB)], wb)

            def chunk(ci, _):
                off = ci * CG
                for q in range(CG // 16):
                    sc[pl.ds(q * 16, 16)] = sb[pl.ds(off + q * 16, 16)]
                    dc[pl.ds(q * 16, 16)] = db[pl.ds(off + q * 16, 16)]
                pltpu.async_copy(tab_hbm.at[sc], rows, sem).wait()

                def edge(i, _):
                    ewv = plsc.load_gather(
                        wb, [jnp.zeros((16,), jnp.int32) + (off + i)])
                    for j in range(H // 16):
                        rows[i, pl.ds(j * 16, 16)] = (
                            rows[i, pl.ds(j * 16, 16)] * ewv)
                    return 0
                lax.fori_loop(0, CG, edge, 0)
                pltpu.sync_copy(rows, acc.at[dc], add=True)
                return 0
            lax.fori_loop(0, NCH, chunk, 0)
            return 0
        lax.fori_loop(0, NBLK, blk, 0)

        plsc.subcore_barrier()
        pltpu.sync_copy(acc.at[pl.ds(sid * SUB_ROWS, SUB_ROWS)],
                        out_hbm.at[cid, pl.ds(sid * SUB_ROWS, SUB_ROWS)])

    return prop(table, src, dst, ew)


_BLK = 1024
_GRID = NPAD // _BLK


def _dot(a, b):
    return jnp.dot(a, b, preferred_element_type=jnp.float32)


def _elu(x):
    return jnp.where(x > 0, x, jnp.exp(jnp.minimum(x, 0.0)) - 1.0)


def _tc_reduce_b(B):
    def body(b_ref, a_ref, dinv_ref, dcinv_ref):
        i = pl.program_id(0)
        blk = b_ref[...]
        part = (blk[0:256] + blk[256:512] + blk[512:768] + blk[768:1024])

        @pl.when(i == 0)
        def _():
            a_ref[...] = jnp.zeros_like(a_ref)
        a_ref[...] += part
        dinv_ref[...] = 1.0 / (jnp.sum(blk, axis=1, keepdims=True) + 1.0)

        @pl.when(i == _GRID - 1)
        def _():
            dcinv_ref[...] = 1.0 / (
                jnp.sum(a_ref[...], axis=1, keepdims=True) + 1.0)

    return pl.pallas_call(
        body,
        grid=(_GRID,),
        in_specs=[pl.BlockSpec((_BLK, K), lambda i: (i, 0))],
        out_specs=[
            pl.BlockSpec((K, K), lambda i: (0, 0)),
            pl.BlockSpec((_BLK, 1), lambda i: (i, 0)),
            pl.BlockSpec((K, 1), lambda i: (0, 0)),
        ],
        out_shape=[
            jax.ShapeDtypeStruct((K, K), jnp.float32),
            jax.ShapeDtypeStruct((NPAD, 1), jnp.float32),
            jax.ShapeDtypeStruct((K, 1), jnp.float32),
        ],
    )(B)


def _tc_embed(xcat, W1, b1, W2, b2):
    def body(x_ref, w1_ref, b1_ref, w2_ref, b2_ref, o_ref):
        h = _elu(_dot(x_ref[...], w1_ref[...]) + b1_ref[...])
        o_ref[...] = _elu(_dot(h, w2_ref[...]) + b2_ref[...])

    return pl.pallas_call(
        body,
        grid=(_GRID,),
        in_specs=[
            pl.BlockSpec((_BLK, 8), lambda i: (i, 0)),
            pl.BlockSpec((8, H), lambda i: (0, 0)),
            pl.BlockSpec((1, H), lambda i: (0, 0)),
            pl.BlockSpec((H, H), lambda i: (0, 0)),
            pl.BlockSpec((1, H), lambda i: (0, 0)),
        ],
        out_specs=pl.BlockSpec((_BLK, H), lambda i: (i, 0)),
        out_shape=jax.ShapeDtypeStruct((NPAD, H), jnp.float32),
    )(xcat, W1, b1, W2, b2)


def _tc_gcn_fin(p, h, deginv, W, b):
    def body(p_ref, h_ref, d_ref, w_ref, b_ref, o_ref):
        z = (p_ref[0] + p_ref[1] + h_ref[...]) * d_ref[...]
        o_ref[...] = _elu(_dot(z, w_ref[...]) + b_ref[...])

    return pl.pallas_call(
        body,
        grid=(_GRID,),
        in_specs=[
            pl.BlockSpec((2, _BLK, H), lambda i: (0, i, 0)),
            pl.BlockSpec((_BLK, H), lambda i: (i, 0)),
            pl.BlockSpec((_BLK, 1), lambda i: (i, 0)),
            pl.BlockSpec((H, H), lambda i: (0, 0)),
            pl.BlockSpec((1, H), lambda i: (0, 0)),
        ],
        out_specs=pl.BlockSpec((_BLK, H), lambda i: (i, 0)),
        out_shape=jax.ShapeDtypeStruct((NPAD, H), jnp.float32),
    )(p, h, deginv, W, b)


def _tc_gcn_pool(p, h, deginv, W, b, cntinv):
    def body(p_ref, h_ref, d_ref, w_ref, b_ref, c_ref, pool_ref):
        i = pl.program_id(0)
        z = (p_ref[0] + p_ref[1] + h_ref[...]) * d_ref[...]
        hf = _elu(_dot(z, w_ref[...]) + b_ref[...])
        gid = i * _BLK + lax.broadcasted_iota(jnp.int32, (_BLK, 1), 0)
        hf = jnp.where(gid < N, hf, 0.0)
        part = hf[0:256] + hf[256:512] + hf[512:768] + hf[768:1024]

        @pl.when(i == 0)
        def _():
            pool_ref[...] = jnp.zeros_like(pool_ref)
        pool_ref[...] += part

        @pl.when(i == _GRID - 1)
        def _():
            pool_ref[...] *= c_ref[...]

    return pl.pallas_call(
        body,
        grid=(_GRID,),
        in_specs=[
            pl.BlockSpec((2, _BLK, H), lambda i: (0, i, 0)),
            pl.BlockSpec((_BLK, H), lambda i: (i, 0)),
            pl.BlockSpec((_BLK, 1), lambda i: (i, 0)),
            pl.BlockSpec((H, H), lambda i: (0, 0)),
            pl.BlockSpec((1, H), lambda i: (0, 0)),
            pl.BlockSpec((K, 1), lambda i: (0, 0)),
        ],
        out_specs=pl.BlockSpec((K, H), lambda i: (0, 0)),
        out_shape=jax.ShapeDtypeStruct((K, H), jnp.float32),
    )(p, h, deginv, W, b, cntinv)


def _tc_lstm(A, degcinv, xin, hprev, cprev, Wx, Wh, bcat, wci, wcf, wco):
    def body(a_ref, dc_ref, x_ref, h_ref, c_ref, wx_ref, wh_ref, b_ref,
             wci_ref, wcf_ref, wco_ref, hn_ref, cn_ref):
        a = a_ref[...]
        dinv = dc_ref[...]
        x = x_ref[...]
        hp = h_ref[...]
        c = c_ref[...]
        xp = (_dot(a, x) + x) * dinv
        hpp = (_dot(a, hp) + hp) * dinv
        g = _dot(xp, wx_ref[...]) + _dot(hpp, wh_ref[...]) + b_ref[...]
        gi, gf, gc, go = g[:, 0:128], g[:, 128:256], g[:, 256:384], g[:, 384:512]
        i = jax.nn.sigmoid(gi + wci_ref[...] * c)
        f = jax.nn.sigmoid(gf + wcf_ref[...] * c)
        cn = f * c + i * jnp.tanh(gc)
        o = jax.nn.sigmoid(go + wco_ref[...] * cn)
        hn_ref[...] = o * jnp.tanh(cn)
        cn_ref[...] = cn

    return pl.pallas_call(
        body,
        out_shape=[
            jax.ShapeDtypeStruct((K, H), jnp.float32),
            jax.ShapeDtypeStruct((K, H), jnp.float32),
        ],
    )(A, degcinv, xin, hprev, cprev, Wx, Wh, bcat, wci, wcf, wco)


def _tc_dec_init(A, degcinv, y, Winit, binit):
    def body(a_ref, dc_ref, y_ref, w_ref, b_ref, o_ref):
        y = y_ref[...]
        z = (_dot(a_ref[...], y) + y) * dc_ref[...]
        o_ref[...] = _elu(_dot(z, w_ref[...]) + b_ref[...])

    return pl.pallas_call(
        body,
        out_shape=jax.ShapeDtypeStruct((K, 2 * H), jnp.float32),
    )(A, degcinv, y, Winit, binit)


def _tc_dec_l0(B, dh, deginv, W, b):
    def body(b_ref, dh_ref, d_ref, w_ref, bb_ref, o_ref):
        dh = dh_ref[...]
        agg = _dot(b_ref[...], dh)
        u = jnp.concatenate([dh, dh, dh, dh], axis=0)
        z = (agg + u) * d_ref[...]
        o_ref[...] = _elu(_dot(z, w_ref[...]) + bb_ref[...])

    return pl.pallas_call(
        body,
        grid=(_GRID,),
        in_specs=[
            pl.BlockSpec((_BLK, K), lambda i: (i, 0)),
            pl.BlockSpec((K, H), lambda i: (0, 0)),
            pl.BlockSpec((_BLK, 1), lambda i: (i, 0)),
            pl.BlockSpec((H, H), lambda i: (0, 0)),
            pl.BlockSpec((1, H), lambda i: (0, 0)),
        ],
        out_specs=pl.BlockSpec((_BLK, H), lambda i: (i, 0)),
        out_shape=jax.ShapeDtypeStruct((NPAD, H), jnp.float32),
    )(B, dh, deginv, W, b)


def _tc_head(p, u, deginv, W, b, Wo1, bo1, Wo2, bo2):
    def body(p_ref, u_ref, d_ref, w_ref, b_ref, w1_ref, b1_ref, w2_ref,
             b2_ref, o_ref):
        z = (p_ref[0] + p_ref[1] + u_ref[...]) * d_ref[...]
        z = _elu(_dot(z, w_ref[...]) + b_ref[...])
        z = _elu(_dot(z, w1_ref[...]) + b1_ref[...])
        o_ref[...] = _dot(z, w2_ref[...]) + b2_ref[...]

    return pl.pallas_call(
        body,
        grid=(_GRID,),
        in_specs=[
            pl.BlockSpec((2, _BLK, H), lambda i: (0, i, 0)),
            pl.BlockSpec((_BLK, H), lambda i: (i, 0)),
            pl.BlockSpec((_BLK, 1), lambda i: (i, 0)),
            pl.BlockSpec((H, H), lambda i: (0, 0)),
            pl.BlockSpec((1, H), lambda i: (0, 0)),
            pl.BlockSpec((H, H), lambda i: (0, 0)),
            pl.BlockSpec((1, H), lambda i: (0, 0)),
            pl.BlockSpec((H, 8), lambda i: (0, 0)),
            pl.BlockSpec((1, 8), lambda i: (0, 0)),
        ],
        out_specs=pl.BlockSpec((_BLK, 8), lambda i: (i, 0)),
        out_shape=jax.ShapeDtypeStruct((NPAD, 8), jnp.float32),
    )(p, u, deginv, W, b, Wo1, bo1, Wo2, bo2)


def kernel(x, seq_len, edge_index, edge_attr, pos, params):
    p = params
    src = edge_index[0].astype(jnp.int32)
    dst = edge_index[1].astype(jnp.int32)
    ew = edge_attr

    B = _sc_build_b(src, dst, ew).reshape(NPAD, K)
    A, deginv, degcinv = _tc_reduce_b(B)

    cnt = np.where(np.arange(K) < (N % K), N // K + 1, N // K)
    cntinv = jnp.asarray(1.0 / cnt, jnp.float32).reshape(K, 1)

    W1p = jnp.pad(p["We1"], ((0, 3), (0, 0)))
    b1 = p["be1"].reshape(1, H)
    W2 = p["We2"]
    b2 = p["be2"].reshape(1, H)

    def gcat(pre, kind):
        return jnp.concatenate(
            [p[f"{pre}_W{kind}_{g}"] for g in ("i", "f", "c", "o")], axis=1)

    Wx_e, Wh_e = gcat("enc", "x"), gcat("enc", "h")
    b_e = jnp.concatenate(
        [p[f"enc_b_{g}"] for g in ("i", "f", "c", "o")]).reshape(1, 4 * H)
    Wx_d, Wh_d = gcat("dec", "x"), gcat("dec", "h")
    b_d = jnp.concatenate(
        [p[f"dec_b_{g}"] for g in ("i", "f", "c", "o")]).reshape(1, 4 * H)
    wci_e = p["enc_wci"].reshape(1, H)
    wcf_e = p["enc_wcf"].reshape(1, H)
    wco_e = p["enc_wco"].reshape(1, H)
    wci_d = p["dec_wci"].reshape(1, H)
    wcf_d = p["dec_wcf"].reshape(1, H)
    wco_d = p["dec_wco"].reshape(1, H)
    binit = p["binit"].reshape(1, 2 * H)
    bo1 = p["bo1"].reshape(1, H)
    Wo2p = jnp.pad(p["Wo2"], ((0, 0), (0, 8 - COUT)))
    bo2p = jnp.pad(p["bo2"], (0, 8 - COUT)).reshape(1, 8)

    T = x.shape[0]
    pos_b = jnp.broadcast_to(pos[None], (T, N, 2))
    xcat = jnp.concatenate(
        [x, pos_b, jnp.zeros((T, N, 3), jnp.float32)], axis=2)
    xcat = jnp.pad(xcat, ((0, 0), (0, NPAD - N), (0, 0)))

    h_t = jnp.zeros((K, H), jnp.float32)
    c_t = jnp.zeros((K, H), jnp.float32)
    for t in range(T):
        hf = _tc_embed(xcat[t], W1p, b1, W2, b2)
        pr = _sc_propagate(hf, src, dst, ew)
        hf = _tc_gcn_fin(pr, hf, deginv, p["Wenc_mp0"],
                         p["benc_mp0"].reshape(1, H))
        pr = _sc_propagate(hf, src, dst, ew)
        pooled = _tc_gcn_pool(pr, hf, deginv, p["Wenc_mp1"],
                              p["benc_mp1"].reshape(1, H), cntinv)
        h_t, c_t = _tc_lstm(A, degcinv, pooled, h_t, c_t,
                            Wx_e, Wh_e, b_e, wci_e, wcf_e, wco_e)

    y = jnp.concatenate([h_t, c_t], axis=1)
    comb = _tc_dec_init(A, degcinv, y, p["Winit"], binit)
    dh0, dc0 = comb[:, :H], comb[:, H:]

    def dec_body(t, carry):
        dh, dc, out = carry
        dh, dc = _tc_lstm(A, degcinv, h_t, dh, dc,
                          Wx_d, Wh_d, b_d, wci_d, wcf_d, wco_d)
        u = _tc_dec_l0(B, dh, deginv, p["Wdec_mp0"],
                       p["bdec_mp0"].reshape(1, H))
        pr = _sc_propagate(u, src, dst, ew)
        o = _tc_head(pr, u, deginv, p["Wdec_mp1"],
                     p["bdec_mp1"].reshape(1, H),
                     p["Wo1"], bo1, Wo2p, bo2p)
        out = lax.dynamic_update_index_in_dim(out, o, t, axis=0)
        return dh, dc, out

    out0 = jnp.zeros((4, NPAD, 8), jnp.float32)
    _, _, out = lax.fori_loop(0, seq_len, dec_body, (dh0, dc0, out0))
    return out[:, :N, :COUT]


# speedup vs baseline: 18.1688x; 1.7465x over previous
import functools

import jax
import jax.numpy as jnp
import numpy as np
from jax import lax
from jax.experimental import pallas as pl
from jax.experimental.pallas import tpu as pltpu
from jax.experimental.pallas import tpu_sc as plsc

N = 10000
E = 320000
K = 256
H = 128
NPAD = 10240
FOLD = NPAD // K
COUT = 3
NMP = 2

NW = 32
EPW = E // NW
ROWS_B = NPAD // NW
SUB_ROWS = NPAD // 16


def _sc_build_b(src, dst, ew):
    mesh = plsc.VectorSubcoreMesh(core_axis_name="c", subcore_axis_name="s")
    CBLK = 2000
    NBLK = E // CBLK

    @functools.partial(
        pl.kernel,
        out_type=jax.ShapeDtypeStruct((NPAD * K,), jnp.float32),
        mesh=mesh,
        compiler_params=pltpu.CompilerParams(needs_layout_passes=False),
        scratch_types=[
            pltpu.VMEM((ROWS_B * K,), jnp.float32),
            pltpu.VMEM((CBLK,), jnp.int32),
            pltpu.VMEM((CBLK,), jnp.int32),
            pltpu.VMEM((CBLK,), jnp.float32),
            pltpu.VMEM((CBLK,), jnp.int32),
            pltpu.VMEM((CBLK,), jnp.int32),
            pltpu.VMEM((CBLK,), jnp.float32),
            pltpu.SemaphoreType.DMA,
            pltpu.SemaphoreType.DMA,
            pltpu.SemaphoreType.DMA,
            pltpu.SemaphoreType.DMA,
            pltpu.SemaphoreType.DMA,
            pltpu.SemaphoreType.DMA,
        ],
    )
    def build(src_hbm, dst_hbm, ew_hbm, b_hbm, bt,
              sb0, db0, wb0, sb1, db1, wb1, s0, s1, s2, s3, s4, s5):
        wid = lax.axis_index("s") * 2 + lax.axis_index("c")
        base = wid * ROWS_B
        zero = jnp.zeros((16,), jnp.float32)

        def zrow(i, _):
            bt[pl.ds(i * 16, 16)] = zero
            return 0
        lax.fori_loop(0, ROWS_B * K // 16, zrow, 0, unroll=8)

        bufs = ((sb0, db0, wb0, s0, s1, s2), (sb1, db1, wb1, s3, s4, s5))

        def fire(b, p):
            sbb, dbb, wbb, ss, sd, sw = bufs[p]
            e0 = b * CBLK
            pltpu.async_copy(src_hbm.at[pl.ds(e0, CBLK)], sbb, ss)
            pltpu.async_copy(dst_hbm.at[pl.ds(e0, CBLK)], dbb, sd)
            pltpu.async_copy(ew_hbm.at[pl.ds(e0, CBLK)], wbb, sw)

        def scan(b, p):
            sbb, dbb, wbb, ss, sd, sw = bufs[p]
            e0 = b * CBLK
            pltpu.make_async_copy(src_hbm.at[pl.ds(e0, CBLK)], sbb, ss).wait()
            pltpu.make_async_copy(dst_hbm.at[pl.ds(e0, CBLK)], dbb, sd).wait()
            pltpu.make_async_copy(ew_hbm.at[pl.ds(e0, CBLK)], wbb, sw).wait()

            def grp(g, _):
                d = dbb[pl.ds(g * 16, 16)]
                s = sbb[pl.ds(g * 16, 16)]
                w = wbb[pl.ds(g * 16, 16)]
                hs = lax.bitwise_and(s, K - 1)
                r = d - base
                m = (d >= base) & (d < base + ROWS_B)
                idx = jnp.where(m, lax.shift_left(r, 8) + hs, 0)
                plsc.addupdate_scatter(bt, [idx], w, mask=m)
                return 0
            lax.fori_loop(0, CBLK // 16, grp, 0, unroll=5)

        fire(0, 0)

        def pairloop(i, _):
            c = 2 * i
            fire(c + 1, 1)
            scan(c, 0)

            @pl.when(i < NBLK // 2 - 1)
            def _():
                fire(c + 2, 0)
            scan(c + 1, 1)
            return 0
        lax.fori_loop(0, NBLK // 2, pairloop, 0)
        pltpu.sync_copy(bt, b_hbm.at[pl.ds(base * K, ROWS_B * K)])

    return build(src, dst, ew)


def _sc_propagate(table, src, dst, ew):
    mesh = plsc.VectorSubcoreMesh(core_axis_name="c", subcore_axis_name="s")
    CG = 80
    NCH = EPW // CG
    NPAIR = (NCH - 1) // 2

    CB = 2000
    CPB = CB // CG

    @functools.partial(
        pl.kernel,
        out_type=jax.ShapeDtypeStruct((2, NPAD, H), jnp.float32),
        mesh=mesh,
        compiler_params=pltpu.CompilerParams(needs_layout_passes=False),
        scratch_types=[
            pltpu.VMEM_SHARED((NPAD, H), jnp.float32),
            pltpu.VMEM((CG, H), jnp.float32),
            pltpu.VMEM((CG, H), jnp.float32),
            pltpu.VMEM((CB,), jnp.int32),
            pltpu.VMEM((CB,), jnp.int32),
            pltpu.VMEM((CB,), jnp.float32),
            pltpu.VMEM((CG,), jnp.int32),
            pltpu.VMEM((CG,), jnp.int32),
            pltpu.VMEM((CG,), jnp.int32),
            pltpu.VMEM((CG,), jnp.int32),
            pltpu.VMEM((CG,), jnp.float32),
            pltpu.VMEM((CG,), jnp.float32),
            pltpu.SemaphoreType.DMA,
            pltpu.SemaphoreType.DMA,
            pltpu.SemaphoreType.DMA,
            pltpu.SemaphoreType.DMA,
        ],
    )
    def prop(tab_hbm, src_hbm, dst_hbm, ew_hbm, out_hbm,
             acc, rows0, rows1, sb, db, wb,
             sc0, sc1, dc0, dc1, wc0, wc1,
             gsem0, gsem1, ssem0, ssem1):
        cid = lax.axis_index("c")
        sid = lax.axis_index("s")
        wid = sid * 2 + cid
        zero = jnp.zeros((16,), jnp.float32)

        def zrow(i, _):
            for j in range(H // 16):
                rows0[i, pl.ds(j * 16, 16)] = zero
            return 0
        lax.fori_loop(0, CG, zrow, 0)

        def zstripe(i, _):
            pltpu.sync_copy(rows0, acc.at[pl.ds(sid * SUB_ROWS + i * CG, CG)])
            return 0
        lax.fori_loop(0, SUB_ROWS // CG, zstripe, 0)
        plsc.subcore_barrier()

        def stage_fire(ci, scb, dcb, wcb, rowsb, gsem):
            bi = lax.div(ci, CPB)
            off = lax.rem(ci, CPB) * CG

            @pl.when(off == 0)
            def _():
                e0 = wid * EPW + bi * CB
                pltpu.sync_copy(src_hbm.at[pl.ds(e0, CB)], sb)
                pltpu.sync_copy(dst_hbm.at[pl.ds(e0, CB)], db)
                pltpu.sync_copy(ew_hbm.at[pl.ds(e0, CB)], wb)
            for q in range(CG // 16):
                scb[pl.ds(q * 16, 16)] = sb[pl.ds(off + q * 16, 16)]
                dcb[pl.ds(q * 16, 16)] = db[pl.ds(off + q * 16, 16)]
                wcb[pl.ds(q * 16, 16)] = wb[pl.ds(off + q * 16, 16)]
            pltpu.async_copy(tab_hbm.at[scb], rowsb, gsem)

        def process(scb, dcb, wcb, rowsb, gsem, ssem):
            pltpu.make_async_copy(tab_hbm.at[scb], rowsb, gsem).wait()

            def edge(i, _):
                ewv = plsc.load_gather(wcb, [jnp.zeros((16,), jnp.int32) + i])
                for j in range(H // 16):
                    rowsb[i, pl.ds(j * 16, 16)] = (
                        rowsb[i, pl.ds(j * 16, 16)] * ewv)
                return 0
            lax.fori_loop(0, CG, edge, 0, unroll=8)
            pltpu.async_copy(rowsb, acc.at[dcb], ssem, add=True)

        def wait_scatter(rowsb, dcb, ssem):
            pltpu.make_async_copy(rowsb, acc.at[dcb], ssem).wait()

        stage_fire(0, sc0, dc0, wc0, rows0, gsem0)

        def pair(i, _):
            c = 2 * i

            @pl.when(i > 0)
            def _():
                wait_scatter(rows1, dc1, ssem1)
            stage_fire(c + 1, sc1, dc1, wc1, rows1, gsem1)
            process(sc0, dc0, wc0, rows0, gsem0, ssem0)
            wait_scatter(rows0, dc0, ssem0)
            stage_fire(c + 2, sc0, dc0, wc0, rows0, gsem0)
            process(sc1, dc1, wc1, rows1, gsem1, ssem1)
            return 0
        lax.fori_loop(0, NPAIR, pair, 0)
        process(sc0, dc0, wc0, rows0, gsem0, ssem0)
        wait_scatter(rows0, dc0, ssem0)
        wait_scatter(rows1, dc1, ssem1)

        plsc.subcore_barrier()
        pltpu.sync_copy(acc.at[pl.ds(sid * SUB_ROWS, SUB_ROWS)],
                        out_hbm.at[cid, pl.ds(sid * SUB_ROWS, SUB_ROWS)])

    return prop(table, src, dst, ew)


_BLK = 1024
_GRID = NPAD // _BLK


def _dot(a, b):
    return jnp.dot(a, b, preferred_element_type=jnp.float32)


def _elu(x):
    return jnp.where(x > 0, x, jnp.exp(jnp.minimum(x, 0.0)) - 1.0)


def _tc_reduce_b(B):
    def body(b_ref, a_ref, dinv_ref, dcinv_ref):
        i = pl.program_id(0)
        blk = b_ref[...]
        part = (blk[0:256] + blk[256:512] + blk[512:768] + blk[768:1024])

        @pl.when(i == 0)
        def _():
            a_ref[...] = jnp.zeros_like(a_ref)
        a_ref[...] += part
        dinv_ref[...] = 1.0 / (jnp.sum(blk, axis=1, keepdims=True) + 1.0)

        @pl.when(i == _GRID - 1)
        def _():
            dcinv_ref[...] = 1.0 / (
                jnp.sum(a_ref[...], axis=1, keepdims=True) + 1.0)

    return pl.pallas_call(
        body,
        grid=(_GRID,),
        in_specs=[pl.BlockSpec((_BLK, K), lambda i: (i, 0))],
        out_specs=[
            pl.BlockSpec((K, K), lambda i: (0, 0)),
            pl.BlockSpec((_BLK, 1), lambda i: (i, 0)),
            pl.BlockSpec((K, 1), lambda i: (0, 0)),
        ],
        out_shape=[
            jax.ShapeDtypeStruct((K, K), jnp.float32),
            jax.ShapeDtypeStruct((NPAD, 1), jnp.float32),
            jax.ShapeDtypeStruct((K, 1), jnp.float32),
        ],
    )(B)


def _tc_embed(xcat, W1, b1, W2, b2):
    def body(x_ref, w1_ref, b1_ref, w2_ref, b2_ref, o_ref):
        h = _elu(_dot(x_ref[...], w1_ref[...]) + b1_ref[...])
        o_ref[...] = _elu(_dot(h, w2_ref[...]) + b2_ref[...])

    return pl.pallas_call(
        body,
        grid=(_GRID,),
        in_specs=[
            pl.BlockSpec((_BLK, 8), lambda i: (i, 0)),
            pl.BlockSpec((8, H), lambda i: (0, 0)),
            pl.BlockSpec((1, H), lambda i: (0, 0)),
            pl.BlockSpec((H, H), lambda i: (0, 0)),
            pl.BlockSpec((1, H), lambda i: (0, 0)),
        ],
        out_specs=pl.BlockSpec((_BLK, H), lambda i: (i, 0)),
        out_shape=jax.ShapeDtypeStruct((NPAD, H), jnp.float32),
    )(xcat, W1, b1, W2, b2)


def _tc_gcn_fin(p, h, deginv, W, b):
    def body(p_ref, h_ref, d_ref, w_ref, b_ref, o_ref):
        z = (p_ref[0] + p_ref[1] + h_ref[...]) * d_ref[...]
        o_ref[...] = _elu(_dot(z, w_ref[...]) + b_ref[...])

    return pl.pallas_call(
        body,
        grid=(_GRID,),
        in_specs=[
            pl.BlockSpec((2, _BLK, H), lambda i: (0, i, 0)),
            pl.BlockSpec((_BLK, H), lambda i: (i, 0)),
            pl.BlockSpec((_BLK, 1), lambda i: (i, 0)),
            pl.BlockSpec((H, H), lambda i: (0, 0)),
            pl.BlockSpec((1, H), lambda i: (0, 0)),
        ],
        out_specs=pl.BlockSpec((_BLK, H), lambda i: (i, 0)),
        out_shape=jax.ShapeDtypeStruct((NPAD, H), jnp.float32),
    )(p, h, deginv, W, b)


def _tc_gcn_pool(p, h, deginv, W, b, cntinv):
    def body(p_ref, h_ref, d_ref, w_ref, b_ref, c_ref, pool_ref):
        i = pl.program_id(0)
        z = (p_ref[0] + p_ref[1] + h_ref[...]) * d_ref[...]
        hf = _elu(_dot(z, w_ref[...]) + b_ref[...])
        gid = i * _BLK + lax.broadcasted_iota(jnp.int32, (_BLK, 1), 0)
        hf = jnp.where(gid < N, hf, 0.0)
        part = hf[0:256] + hf[256:512] + hf[512:768] + hf[768:1024]

        @pl.when(i == 0)
        def _():
            pool_ref[...] = jnp.zeros_like(pool_ref)
        pool_ref[...] += part

        @pl.when(i == _GRID - 1)
        def _():
            pool_ref[...] *= c_ref[...]

    return pl.pallas_call(
        body,
        grid=(_GRID,),
        in_specs=[
            pl.BlockSpec((2, _BLK, H), lambda i: (0, i, 0)),
            pl.BlockSpec((_BLK, H), lambda i: (i, 0)),
            pl.BlockSpec((_BLK, 1), lambda i: (i, 0)),
            pl.BlockSpec((H, H), lambda i: (0, 0)),
            pl.BlockSpec((1, H), lambda i: (0, 0)),
            pl.BlockSpec((K, 1), lambda i: (0, 0)),
        ],
        out_specs=pl.BlockSpec((K, H), lambda i: (0, 0)),
        out_shape=jax.ShapeDtypeStruct((K, H), jnp.float32),
    )(p, h, deginv, W, b, cntinv)


def _tc_lstm(A, degcinv, xin, hprev, cprev, Wx, Wh, bcat, wci, wcf, wco):
    def body(a_ref, dc_ref, x_ref, h_ref, c_ref, wx_ref, wh_ref, b_ref,
             wci_ref, wcf_ref, wco_ref, hn_ref, cn_ref):
        a = a_ref[...]
        dinv = dc_ref[...]
        x = x_ref[...]
        hp = h_ref[...]
        c = c_ref[...]
        xp = (_dot(a, x) + x) * dinv
        hpp = (_dot(a, hp) + hp) * dinv
        g = _dot(xp, wx_ref[...]) + _dot(hpp, wh_ref[...]) + b_ref[...]
        gi, gf, gc, go = g[:, 0:128], g[:, 128:256], g[:, 256:384], g[:, 384:512]
        i = jax.nn.sigmoid(gi + wci_ref[...] * c)
        f = jax.nn.sigmoid(gf + wcf_ref[...] * c)
        cn = f * c + i * jnp.tanh(gc)
        o = jax.nn.sigmoid(go + wco_ref[...] * cn)
        hn_ref[...] = o * jnp.tanh(cn)
        cn_ref[...] = cn

    return pl.pallas_call(
        body,
        out_shape=[
            jax.ShapeDtypeStruct((K, H), jnp.float32),
            jax.ShapeDtypeStruct((K, H), jnp.float32),
        ],
    )(A, degcinv, xin, hprev, cprev, Wx, Wh, bcat, wci, wcf, wco)


def _tc_dec_init(A, degcinv, y, Winit, binit):
    def body(a_ref, dc_ref, y_ref, w_ref, b_ref, o_ref):
        y = y_ref[...]
        z = (_dot(a_ref[...], y) + y) * dc_ref[...]
        o_ref[...] = _elu(_dot(z, w_ref[...]) + b_ref[...])

    return pl.pallas_call(
        body,
        out_shape=jax.ShapeDtypeStruct((K, 2 * H), jnp.float32),
    )(A, degcinv, y, Winit, binit)


def _tc_dec_l0(B, dh, deginv, W, b):
    def body(b_ref, dh_ref, d_ref, w_ref, bb_ref, o_ref):
        dh = dh_ref[...]
        agg = _dot(b_ref[...], dh)
        u = jnp.concatenate([dh, dh, dh, dh], axis=0)
        z = (agg + u) * d_ref[...]
        o_ref[...] = _elu(_dot(z, w_ref[...]) + bb_ref[...])

    return pl.pallas_call(
        body,
        grid=(_GRID,),
        in_specs=[
            pl.BlockSpec((_BLK, K), lambda i: (i, 0)),
            pl.BlockSpec((K, H), lambda i: (0, 0)),
            pl.BlockSpec((_BLK, 1), lambda i: (i, 0)),
            pl.BlockSpec((H, H), lambda i: (0, 0)),
            pl.BlockSpec((1, H), lambda i: (0, 0)),
        ],
        out_specs=pl.BlockSpec((_BLK, H), lambda i: (i, 0)),
        out_shape=jax.ShapeDtypeStruct((NPAD, H), jnp.float32),
    )(B, dh, deginv, W, b)


def _tc_head(p, u, deginv, W, b, Wo1, bo1, Wo2, bo2):
    def body(p_ref, u_ref, d_ref, w_ref, b_ref, w1_ref, b1_ref, w2_ref,
             b2_ref, o_ref):
        z = (p_ref[0] + p_ref[1] + u_ref[...]) * d_ref[...]
        z = _elu(_dot(z, w_ref[...]) + b_ref[...])
        z = _elu(_dot(z, w1_ref[...]) + b1_ref[...])
        o_ref[...] = _dot(z, w2_ref[...]) + b2_ref[...]

    return pl.pallas_call(
        body,
        grid=(_GRID,),
        in_specs=[
            pl.BlockSpec((2, _BLK, H), lambda i: (0, i, 0)),
            pl.BlockSpec((_BLK, H), lambda i: (i, 0)),
            pl.BlockSpec((_BLK, 1), lambda i: (i, 0)),
            pl.BlockSpec((H, H), lambda i: (0, 0)),
            pl.BlockSpec((1, H), lambda i: (0, 0)),
            pl.BlockSpec((H, H), lambda i: (0, 0)),
            pl.BlockSpec((1, H), lambda i: (0, 0)),
            pl.BlockSpec((H, 8), lambda i: (0, 0)),
            pl.BlockSpec((1, 8), lambda i: (0, 0)),
        ],
        out_specs=pl.BlockSpec((_BLK, 8), lambda i: (i, 0)),
        out_shape=jax.ShapeDtypeStruct((NPAD, 8), jnp.float32),
    )(p, u, deginv, W, b, Wo1, bo1, Wo2, bo2)


def kernel(x, seq_len, edge_index, edge_attr, pos, params):
    p = params
    src = edge_index[0].astype(jnp.int32)
    dst = edge_index[1].astype(jnp.int32)
    ew = edge_attr

    B = _sc_build_b(src, dst, ew).reshape(NPAD, K)
    A, deginv, degcinv = _tc_reduce_b(B)

    cnt = np.where(np.arange(K) < (N % K), N // K + 1, N // K)
    cntinv = jnp.asarray(1.0 / cnt, jnp.float32).reshape(K, 1)

    W1p = jnp.pad(p["We1"], ((0, 3), (0, 0)))
    b1 = p["be1"].reshape(1, H)
    W2 = p["We2"]
    b2 = p["be2"].reshape(1, H)

    def gcat(pre, kind):
        return jnp.concatenate(
            [p[f"{pre}_W{kind}_{g}"] for g in ("i", "f", "c", "o")], axis=1)

    Wx_e, Wh_e = gcat("enc", "x"), gcat("enc", "h")
    b_e = jnp.concatenate(
        [p[f"enc_b_{g}"] for g in ("i", "f", "c", "o")]).reshape(1, 4 * H)
    Wx_d, Wh_d = gcat("dec", "x"), gcat("dec", "h")
    b_d = jnp.concatenate(
        [p[f"dec_b_{g}"] for g in ("i", "f", "c", "o")]).reshape(1, 4 * H)
    wci_e = p["enc_wci"].reshape(1, H)
    wcf_e = p["enc_wcf"].reshape(1, H)
    wco_e = p["enc_wco"].reshape(1, H)
    wci_d = p["dec_wci"].reshape(1, H)
    wcf_d = p["dec_wcf"].reshape(1, H)
    wco_d = p["dec_wco"].reshape(1, H)
    binit = p["binit"].reshape(1, 2 * H)
    bo1 = p["bo1"].reshape(1, H)
    Wo2p = jnp.pad(p["Wo2"], ((0, 0), (0, 8 - COUT)))
    bo2p = jnp.pad(p["bo2"], (0, 8 - COUT)).reshape(1, 8)

    T = x.shape[0]
    pos_b = jnp.broadcast_to(pos[None], (T, N, 2))
    xcat = jnp.concatenate(
        [x, pos_b, jnp.zeros((T, N, 3), jnp.float32)], axis=2)
    xcat = jnp.pad(xcat, ((0, 0), (0, NPAD - N), (0, 0)))

    h_t = jnp.zeros((K, H), jnp.float32)
    c_t = jnp.zeros((K, H), jnp.float32)
    for t in range(T):
        hf = _tc_embed(xcat[t], W1p, b1, W2, b2)
        pr = _sc_propagate(hf, src, dst, ew)
        hf = _tc_gcn_fin(pr, hf, deginv, p["Wenc_mp0"],
                         p["benc_mp0"].reshape(1, H))
        pr = _sc_propagate(hf, src, dst, ew)
        pooled = _tc_gcn_pool(pr, hf, deginv, p["Wenc_mp1"],
                              p["benc_mp1"].reshape(1, H), cntinv)
        h_t, c_t = _tc_lstm(A, degcinv, pooled, h_t, c_t,
                            Wx_e, Wh_e, b_e, wci_e, wcf_e, wco_e)

    y = jnp.concatenate([h_t, c_t], axis=1)
    comb = _tc_dec_init(A, degcinv, y, p["Winit"], binit)
    dh0, dc0 = comb[:, :H], comb[:, H:]

    def dec_body(t, carry):
        dh, dc, out = carry
        dh, dc = _tc_lstm(A, degcinv, h_t, dh, dc,
                          Wx_d, Wh_d, b_d, wci_d, wcf_d, wco_d)
        u = _tc_dec_l0(B, dh, deginv, p["Wdec_mp0"],
                       p["bdec_mp0"].reshape(1, H))
        pr = _sc_propagate(u, src, dst, ew)
        o = _tc_head(pr, u, deginv, p["Wdec_mp1"],
                     p["bdec_mp1"].reshape(1, H),
                     p["Wo1"], bo1, Wo2p, bo2p)
        out = lax.dynamic_update_index_in_dim(out, o, t, axis=0)
        return dh, dc, out

    out0 = jnp.zeros((4, NPAD, 8), jnp.float32)
    _, _, out = lax.fori_loop(0, seq_len, dec_body, (dh0, dc0, out0))
    return out[:, :N, :COUT]


# speedup vs baseline: 20.9120x; 1.1510x over previous
import functools

import jax
import jax.numpy as jnp
import numpy as np
from jax import lax
from jax.experimental import pallas as pl
from jax.experimental.pallas import tpu as pltpu
from jax.experimental.pallas import tpu_sc as plsc

N = 10000
E = 320000
K = 256
H = 128
NPAD = 10240
FOLD = NPAD // K
COUT = 3
NMP = 2

NW = 32
EPW = E // NW
ROWS_B = NPAD // NW
SUB_ROWS = NPAD // 16


def _sc_build_b(src, dst, ew):
    mesh = plsc.VectorSubcoreMesh(core_axis_name="c", subcore_axis_name="s")
    CBLK = 2000
    NBLK = E // CBLK

    @functools.partial(
        pl.kernel,
        out_type=jax.ShapeDtypeStruct((NPAD * K,), jnp.float32),
        mesh=mesh,
        compiler_params=pltpu.CompilerParams(needs_layout_passes=False),
        scratch_types=[
            pltpu.VMEM((ROWS_B * K,), jnp.float32),
            pltpu.VMEM((CBLK,), jnp.int32),
            pltpu.VMEM((CBLK,), jnp.int32),
            pltpu.VMEM((CBLK,), jnp.float32),
            pltpu.VMEM((CBLK,), jnp.int32),
            pltpu.VMEM((CBLK,), jnp.int32),
            pltpu.VMEM((CBLK,), jnp.float32),
            pltpu.SemaphoreType.DMA,
            pltpu.SemaphoreType.DMA,
            pltpu.SemaphoreType.DMA,
            pltpu.SemaphoreType.DMA,
            pltpu.SemaphoreType.DMA,
            pltpu.SemaphoreType.DMA,
        ],
    )
    def build(src_hbm, dst_hbm, ew_hbm, b_hbm, bt,
              sb0, db0, wb0, sb1, db1, wb1, s0, s1, s2, s3, s4, s5):
        wid = lax.axis_index("s") * 2 + lax.axis_index("c")
        base = wid * ROWS_B
        zero = jnp.zeros((16,), jnp.float32)

        def zrow(i, _):
            bt[pl.ds(i * 16, 16)] = zero
            return 0
        lax.fori_loop(0, ROWS_B * K // 16, zrow, 0, unroll=8)

        bufs = ((sb0, db0, wb0, s0, s1, s2), (sb1, db1, wb1, s3, s4, s5))

        def fire(b, p):
            sbb, dbb, wbb, ss, sd, sw = bufs[p]
            e0 = b * CBLK
            pltpu.async_copy(src_hbm.at[pl.ds(e0, CBLK)], sbb, ss)
            pltpu.async_copy(dst_hbm.at[pl.ds(e0, CBLK)], dbb, sd)
            pltpu.async_copy(ew_hbm.at[pl.ds(e0, CBLK)], wbb, sw)

        def scan(b, p):
            sbb, dbb, wbb, ss, sd, sw = bufs[p]
            e0 = b * CBLK
            pltpu.make_async_copy(src_hbm.at[pl.ds(e0, CBLK)], sbb, ss).wait()
            pltpu.make_async_copy(dst_hbm.at[pl.ds(e0, CBLK)], dbb, sd).wait()
            pltpu.make_async_copy(ew_hbm.at[pl.ds(e0, CBLK)], wbb, sw).wait()

            def grp(g, _):
                d = dbb[pl.ds(g * 16, 16)]
                s = sbb[pl.ds(g * 16, 16)]
                w = wbb[pl.ds(g * 16, 16)]
                hs = lax.bitwise_and(s, K - 1)
                r = d - base
                m = (d >= base) & (d < base + ROWS_B)
                idx = jnp.where(m, lax.shift_left(r, 8) + hs, 0)
                plsc.addupdate_scatter(bt, [idx], w, mask=m)
                return 0
            lax.fori_loop(0, CBLK // 16, grp, 0, unroll=5)

        fire(0, 0)

        def pairloop(i, _):
            c = 2 * i
            fire(c + 1, 1)
            scan(c, 0)

            @pl.when(i < NBLK // 2 - 1)
            def _():
                fire(c + 2, 0)
            scan(c + 1, 1)
            return 0
        lax.fori_loop(0, NBLK // 2, pairloop, 0)
        pltpu.sync_copy(bt, b_hbm.at[pl.ds(base * K, ROWS_B * K)])

    return build(src, dst, ew)


def _sc_propagate(table, src, dst, ew):
    mesh = plsc.VectorSubcoreMesh(core_axis_name="c", subcore_axis_name="s")
    CG = 80
    NCH = EPW // CG
    NTRI = (NCH - 2) // 3

    CB = 2000
    CPB = CB // CG

    @functools.partial(
        pl.kernel,
        out_type=jax.ShapeDtypeStruct((2, NPAD, H), jnp.float32),
        mesh=mesh,
        compiler_params=pltpu.CompilerParams(needs_layout_passes=False),
        scratch_types=[
            pltpu.VMEM_SHARED((NPAD, H), jnp.float32),
            pltpu.VMEM((CG, H), jnp.float32),
            pltpu.VMEM((CG, H), jnp.float32),
            pltpu.VMEM((CB,), jnp.int32),
            pltpu.VMEM((CB,), jnp.int32),
            pltpu.VMEM((CB,), jnp.float32),
            pltpu.VMEM((CG, H), jnp.float32),
            pltpu.VMEM((CG,), jnp.int32),
            pltpu.VMEM((CG,), jnp.int32),
            pltpu.VMEM((CG,), jnp.int32),
            pltpu.VMEM((CG,), jnp.int32),
            pltpu.VMEM((CG,), jnp.int32),
            pltpu.VMEM((CG,), jnp.int32),
            pltpu.VMEM((CG,), jnp.float32),
            pltpu.VMEM((CG,), jnp.float32),
            pltpu.VMEM((CG,), jnp.float32),
            pltpu.SemaphoreType.DMA,
            pltpu.SemaphoreType.DMA,
            pltpu.SemaphoreType.DMA,
            pltpu.SemaphoreType.DMA,
            pltpu.SemaphoreType.DMA,
            pltpu.SemaphoreType.DMA,
        ],
    )
    def prop(tab_hbm, src_hbm, dst_hbm, ew_hbm, out_hbm,
             acc, rows0, rows1, sb, db, wb, rows2,
             sc0, sc1, sc2, dc0, dc1, dc2, wc0, wc1, wc2,
             gsem0, gsem1, gsem2, ssem0, ssem1, ssem2):
        cid = lax.axis_index("c")
        sid = lax.axis_index("s")
        wid = sid * 2 + cid
        zero = jnp.zeros((16,), jnp.float32)

        def zrow(i, _):
            for j in range(H // 16):
                rows0[i, pl.ds(j * 16, 16)] = zero
            return 0
        lax.fori_loop(0, CG, zrow, 0)

        def zstripe(i, _):
            pltpu.sync_copy(rows0, acc.at[pl.ds(sid * SUB_ROWS + i * CG, CG)])
            return 0
        lax.fori_loop(0, SUB_ROWS // CG, zstripe, 0)
        plsc.subcore_barrier()

        def stage_fire(ci, scb, dcb, wcb, rowsb, gsem):
            bi = lax.div(ci, CPB)
            off = lax.rem(ci, CPB) * CG

            @pl.when(off == 0)
            def _():
                e0 = wid * EPW + bi * CB
                pltpu.sync_copy(src_hbm.at[pl.ds(e0, CB)], sb)
                pltpu.sync_copy(dst_hbm.at[pl.ds(e0, CB)], db)
                pltpu.sync_copy(ew_hbm.at[pl.ds(e0, CB)], wb)
            for q in range(CG // 16):
                scb[pl.ds(q * 16, 16)] = sb[pl.ds(off + q * 16, 16)]
                dcb[pl.ds(q * 16, 16)] = db[pl.ds(off + q * 16, 16)]
                wcb[pl.ds(q * 16, 16)] = wb[pl.ds(off + q * 16, 16)]
            pltpu.async_copy(tab_hbm.at[scb], rowsb, gsem)

        def process(scb, dcb, wcb, rowsb, gsem, ssem):
            pltpu.make_async_copy(tab_hbm.at[scb], rowsb, gsem).wait()

            def edge(i, _):
                ewv = plsc.load_gather(wcb, [jnp.zeros((16,), jnp.int32) + i])
                for j in range(H // 16):
                    rowsb[i, pl.ds(j * 16, 16)] = (
                        rowsb[i, pl.ds(j * 16, 16)] * ewv)
                return 0
            lax.fori_loop(0, CG, edge, 0, unroll=8)
            pltpu.async_copy(rowsb, acc.at[dcb], ssem, add=True)

        def wait_scatter(rowsb, dcb, ssem):
            pltpu.make_async_copy(rowsb, acc.at[dcb], ssem).wait()

        bufs = ((sc0, dc0, wc0, rows0, gsem0, ssem0),
                (sc1, dc1, wc1, rows1, gsem1, ssem1),
                (sc2, dc2, wc2, rows2, gsem2, ssem2))

        stage_fire(0, *bufs[0][:5])
        stage_fire(1, *bufs[1][:5])

        def triple(i, _):
            for k in range(3):
                c = 3 * i + k
                scb, dcb, wcb, rowsb, gsem, ssem = bufs[k]
                nscb, ndcb, nwcb, nrowsb, ngsem, nssem = bufs[(k + 2) % 3]
                process(scb, dcb, wcb, rowsb, gsem, ssem)

                @pl.when(c >= 1)
                def _():
                    wait_scatter(nrowsb, ndcb, nssem)
                stage_fire(c + 2, nscb, ndcb, nwcb, nrowsb, ngsem)
            return 0
        lax.fori_loop(0, NTRI, triple, 0)
        process(*bufs[0])
        process(*bufs[1])
        wait_scatter(rows0, dc0, ssem0)
        wait_scatter(rows1, dc1, ssem1)
        wait_scatter(rows2, dc2, ssem2)

        plsc.subcore_barrier()
        pltpu.sync_copy(acc.at[pl.ds(sid * SUB_ROWS, SUB_ROWS)],
                        out_hbm.at[cid, pl.ds(sid * SUB_ROWS, SUB_ROWS)])

    return prop(table, src, dst, ew)


_BLK = 1024
_GRID = NPAD // _BLK


def _dot(a, b):
    return jnp.dot(a, b, preferred_element_type=jnp.float32)


def _elu(x):
    return jnp.where(x > 0, x, jnp.exp(jnp.minimum(x, 0.0)) - 1.0)


def _tc_reduce_b(B):
    def body(b_ref, a_ref, dinv_ref, dcinv_ref):
        i = pl.program_id(0)
        blk = b_ref[...]
        part = (blk[0:256] + blk[256:512] + blk[512:768] + blk[768:1024])

        @pl.when(i == 0)
        def _():
            a_ref[...] = jnp.zeros_like(a_ref)
        a_ref[...] += part
        dinv_ref[...] = 1.0 / (jnp.sum(blk, axis=1, keepdims=True) + 1.0)

        @pl.when(i == _GRID - 1)
        def _():
            dcinv_ref[...] = 1.0 / (
                jnp.sum(a_ref[...], axis=1, keepdims=True) + 1.0)

    return pl.pallas_call(
        body,
        grid=(_GRID,),
        in_specs=[pl.BlockSpec((_BLK, K), lambda i: (i, 0))],
        out_specs=[
            pl.BlockSpec((K, K), lambda i: (0, 0)),
            pl.BlockSpec((_BLK, 1), lambda i: (i, 0)),
            pl.BlockSpec((K, 1), lambda i: (0, 0)),
        ],
        out_shape=[
            jax.ShapeDtypeStruct((K, K), jnp.float32),
            jax.ShapeDtypeStruct((NPAD, 1), jnp.float32),
            jax.ShapeDtypeStruct((K, 1), jnp.float32),
        ],
    )(B)


def _tc_embed(xcat, W1, b1, W2, b2):
    def body(x_ref, w1_ref, b1_ref, w2_ref, b2_ref, o_ref):
        h = _elu(_dot(x_ref[...], w1_ref[...]) + b1_ref[...])
        o_ref[...] = _elu(_dot(h, w2_ref[...]) + b2_ref[...])

    return pl.pallas_call(
        body,
        grid=(_GRID,),
        in_specs=[
            pl.BlockSpec((_BLK, 8), lambda i: (i, 0)),
            pl.BlockSpec((8, H), lambda i: (0, 0)),
            pl.BlockSpec((1, H), lambda i: (0, 0)),
            pl.BlockSpec((H, H), lambda i: (0, 0)),
            pl.BlockSpec((1, H), lambda i: (0, 0)),
        ],
        out_specs=pl.BlockSpec((_BLK, H), lambda i: (i, 0)),
        out_shape=jax.ShapeDtypeStruct((NPAD, H), jnp.float32),
    )(xcat, W1, b1, W2, b2)


def _tc_gcn_fin(p, h, deginv, W, b):
    def body(p_ref, h_ref, d_ref, w_ref, b_ref, o_ref):
        z = (p_ref[0] + p_ref[1] + h_ref[...]) * d_ref[...]
        o_ref[...] = _elu(_dot(z, w_ref[...]) + b_ref[...])

    return pl.pallas_call(
        body,
        grid=(_GRID,),
        in_specs=[
            pl.BlockSpec((2, _BLK, H), lambda i: (0, i, 0)),
            pl.BlockSpec((_BLK, H), lambda i: (i, 0)),
            pl.BlockSpec((_BLK, 1), lambda i: (i, 0)),
            pl.BlockSpec((H, H), lambda i: (0, 0)),
            pl.BlockSpec((1, H), lambda i: (0, 0)),
        ],
        out_specs=pl.BlockSpec((_BLK, H), lambda i: (i, 0)),
        out_shape=jax.ShapeDtypeStruct((NPAD, H), jnp.float32),
    )(p, h, deginv, W, b)


def _tc_gcn_pool(p, h, deginv, W, b, cntinv):
    def body(p_ref, h_ref, d_ref, w_ref, b_ref, c_ref, pool_ref):
        i = pl.program_id(0)
        z = (p_ref[0] + p_ref[1] + h_ref[...]) * d_ref[...]
        hf = _elu(_dot(z, w_ref[...]) + b_ref[...])
        gid = i * _BLK + lax.broadcasted_iota(jnp.int32, (_BLK, 1), 0)
        hf = jnp.where(gid < N, hf, 0.0)
        part = hf[0:256] + hf[256:512] + hf[512:768] + hf[768:1024]

        @pl.when(i == 0)
        def _():
            pool_ref[...] = jnp.zeros_like(pool_ref)
        pool_ref[...] += part

        @pl.when(i == _GRID - 1)
        def _():
            pool_ref[...] *= c_ref[...]

    return pl.pallas_call(
        body,
        grid=(_GRID,),
        in_specs=[
            pl.BlockSpec((2, _BLK, H), lambda i: (0, i, 0)),
            pl.BlockSpec((_BLK, H), lambda i: (i, 0)),
            pl.BlockSpec((_BLK, 1), lambda i: (i, 0)),
            pl.BlockSpec((H, H), lambda i: (0, 0)),
            pl.BlockSpec((1, H), lambda i: (0, 0)),
            pl.BlockSpec((K, 1), lambda i: (0, 0)),
        ],
        out_specs=pl.BlockSpec((K, H), lambda i: (0, 0)),
        out_shape=jax.ShapeDtypeStruct((K, H), jnp.float32),
    )(p, h, deginv, W, b, cntinv)


def _tc_lstm(A, degcinv, xin, hprev, cprev, Wx, Wh, bcat, wci, wcf, wco):
    def body(a_ref, dc_ref, x_ref, h_ref, c_ref, wx_ref, wh_ref, b_ref,
             wci_ref, wcf_ref, wco_ref, hn_ref, cn_ref):
        a = a_ref[...]
        dinv = dc_ref[...]
        x = x_ref[...]
        hp = h_ref[...]
        c = c_ref[...]
        xp = (_dot(a, x) + x) * dinv
        hpp = (_dot(a, hp) + hp) * dinv
        g = _dot(xp, wx_ref[...]) + _dot(hpp, wh_ref[...]) + b_ref[...]
        gi, gf, gc, go = g[:, 0:128], g[:, 128:256], g[:, 256:384], g[:, 384:512]
        i = jax.nn.sigmoid(gi + wci_ref[...] * c)
        f = jax.nn.sigmoid(gf + wcf_ref[...] * c)
        cn = f * c + i * jnp.tanh(gc)
        o = jax.nn.sigmoid(go + wco_ref[...] * cn)
        hn_ref[...] = o * jnp.tanh(cn)
        cn_ref[...] = cn

    return pl.pallas_call(
        body,
        out_shape=[
            jax.ShapeDtypeStruct((K, H), jnp.float32),
            jax.ShapeDtypeStruct((K, H), jnp.float32),
        ],
    )(A, degcinv, xin, hprev, cprev, Wx, Wh, bcat, wci, wcf, wco)


def _tc_dec_init(A, degcinv, y, Winit, binit):
    def body(a_ref, dc_ref, y_ref, w_ref, b_ref, o_ref):
        y = y_ref[...]
        z = (_dot(a_ref[...], y) + y) * dc_ref[...]
        o_ref[...] = _elu(_dot(z, w_ref[...]) + b_ref[...])

    return pl.pallas_call(
        body,
        out_shape=jax.ShapeDtypeStruct((K, 2 * H), jnp.float32),
    )(A, degcinv, y, Winit, binit)


def _tc_dec_l0(B, dh, deginv, W, b):
    def body(b_ref, dh_ref, d_ref, w_ref, bb_ref, o_ref):
        dh = dh_ref[...]
        agg = _dot(b_ref[...], dh)
        u = jnp.concatenate([dh, dh, dh, dh], axis=0)
        z = (agg + u) * d_ref[...]
        o_ref[...] = _elu(_dot(z, w_ref[...]) + bb_ref[...])

    return pl.pallas_call(
        body,
        grid=(_GRID,),
        in_specs=[
            pl.BlockSpec((_BLK, K), lambda i: (i, 0)),
            pl.BlockSpec((K, H), lambda i: (0, 0)),
            pl.BlockSpec((_BLK, 1), lambda i: (i, 0)),
            pl.BlockSpec((H, H), lambda i: (0, 0)),
            pl.BlockSpec((1, H), lambda i: (0, 0)),
        ],
        out_specs=pl.BlockSpec((_BLK, H), lambda i: (i, 0)),
        out_shape=jax.ShapeDtypeStruct((NPAD, H), jnp.float32),
    )(B, dh, deginv, W, b)


def _tc_head(p, u, deginv, W, b, Wo1, bo1, Wo2, bo2):
    def body(p_ref, u_ref, d_ref, w_ref, b_ref, w1_ref, b1_ref, w2_ref,
             b2_ref, o_ref):
        z = (p_ref[0] + p_ref[1] + u_ref[...]) * d_ref[...]
        z = _elu(_dot(z, w_ref[...]) + b_ref[...])
        z = _elu(_dot(z, w1_ref[...]) + b1_ref[...])
        o_ref[...] = _dot(z, w2_ref[...]) + b2_ref[...]

    return pl.pallas_call(
        body,
        grid=(_GRID,),
        in_specs=[
            pl.BlockSpec((2, _BLK, H), lambda i: (0, i, 0)),
            pl.BlockSpec((_BLK, H), lambda i: (i, 0)),
            pl.BlockSpec((_BLK, 1), lambda i: (i, 0)),
            pl.BlockSpec((H, H), lambda i: (0, 0)),
            pl.BlockSpec((1, H), lambda i: (0, 0)),
            pl.BlockSpec((H, H), lambda i: (0, 0)),
            pl.BlockSpec((1, H), lambda i: (0, 0)),
            pl.BlockSpec((H, 8), lambda i: (0, 0)),
            pl.BlockSpec((1, 8), lambda i: (0, 0)),
        ],
        out_specs=pl.BlockSpec((_BLK, 8), lambda i: (i, 0)),
        out_shape=jax.ShapeDtypeStruct((NPAD, 8), jnp.float32),
    )(p, u, deginv, W, b, Wo1, bo1, Wo2, bo2)


def kernel(x, seq_len, edge_index, edge_attr, pos, params):
    p = params
    src = edge_index[0].astype(jnp.int32)
    dst = edge_index[1].astype(jnp.int32)
    ew = edge_attr

    B = _sc_build_b(src, dst, ew).reshape(NPAD, K)
    A, deginv, degcinv = _tc_reduce_b(B)

    cnt = np.where(np.arange(K) < (N % K), N // K + 1, N // K)
    cntinv = jnp.asarray(1.0 / cnt, jnp.float32).reshape(K, 1)

    W1p = jnp.pad(p["We1"], ((0, 3), (0, 0)))
    b1 = p["be1"].reshape(1, H)
    W2 = p["We2"]
    b2 = p["be2"].reshape(1, H)

    def gcat(pre, kind):
        return jnp.concatenate(
            [p[f"{pre}_W{kind}_{g}"] for g in ("i", "f", "c", "o")], axis=1)

    Wx_e, Wh_e = gcat("enc", "x"), gcat("enc", "h")
    b_e = jnp.concatenate(
        [p[f"enc_b_{g}"] for g in ("i", "f", "c", "o")]).reshape(1, 4 * H)
    Wx_d, Wh_d = gcat("dec", "x"), gcat("dec", "h")
    b_d = jnp.concatenate(
        [p[f"dec_b_{g}"] for g in ("i", "f", "c", "o")]).reshape(1, 4 * H)
    wci_e = p["enc_wci"].reshape(1, H)
    wcf_e = p["enc_wcf"].reshape(1, H)
    wco_e = p["enc_wco"].reshape(1, H)
    wci_d = p["dec_wci"].reshape(1, H)
    wcf_d = p["dec_wcf"].reshape(1, H)
    wco_d = p["dec_wco"].reshape(1, H)
    binit = p["binit"].reshape(1, 2 * H)
    bo1 = p["bo1"].reshape(1, H)
    Wo2p = jnp.pad(p["Wo2"], ((0, 0), (0, 8 - COUT)))
    bo2p = jnp.pad(p["bo2"], (0, 8 - COUT)).reshape(1, 8)

    T = x.shape[0]
    pos_b = jnp.broadcast_to(pos[None], (T, N, 2))
    xcat = jnp.concatenate(
        [x, pos_b, jnp.zeros((T, N, 3), jnp.float32)], axis=2)
    xcat = jnp.pad(xcat, ((0, 0), (0, NPAD - N), (0, 0)))

    h_t = jnp.zeros((K, H), jnp.float32)
    c_t = jnp.zeros((K, H), jnp.float32)
    for t in range(T):
        hf = _tc_embed(xcat[t], W1p, b1, W2, b2)
        pr = _sc_propagate(hf, src, dst, ew)
        hf = _tc_gcn_fin(pr, hf, deginv, p["Wenc_mp0"],
                         p["benc_mp0"].reshape(1, H))
        pr = _sc_propagate(hf, src, dst, ew)
        pooled = _tc_gcn_pool(pr, hf, deginv, p["Wenc_mp1"],
                              p["benc_mp1"].reshape(1, H), cntinv)
        h_t, c_t = _tc_lstm(A, degcinv, pooled, h_t, c_t,
                            Wx_e, Wh_e, b_e, wci_e, wcf_e, wco_e)

    y = jnp.concatenate([h_t, c_t], axis=1)
    comb = _tc_dec_init(A, degcinv, y, p["Winit"], binit)
    dh0, dc0 = comb[:, :H], comb[:, H:]

    def dec_body(t, carry):
        dh, dc, out = carry
        dh, dc = _tc_lstm(A, degcinv, h_t, dh, dc,
                          Wx_d, Wh_d, b_d, wci_d, wcf_d, wco_d)
        u = _tc_dec_l0(B, dh, deginv, p["Wdec_mp0"],
                       p["bdec_mp0"].reshape(1, H))
        pr = _sc_propagate(u, src, dst, ew)
        o = _tc_head(pr, u, deginv, p["Wdec_mp1"],
                     p["bdec_mp1"].reshape(1, H),
                     p["Wo1"], bo1, Wo2p, bo2p)
        out = lax.dynamic_update_index_in_dim(out, o, t, axis=0)
        return dh, dc, out

    out0 = jnp.zeros((4, NPAD, 8), jnp.float32)
    _, _, out = lax.fori_loop(0, seq_len, dec_body, (dh0, dc0, out0))
    return out[:, :N, :COUT]


# speedup vs baseline: 26.6977x; 1.2767x over previous
import functools

import jax
import jax.numpy as jnp
import numpy as np
from jax import lax
from jax.experimental import pallas as pl
from jax.experimental.pallas import tpu as pltpu
from jax.experimental.pallas import tpu_sc as plsc

N = 10000
E = 320000
K = 256
H = 128
NPAD = 10240
FOLD = NPAD // K
COUT = 3
NMP = 2

NW = 32
EPW = E // NW
ROWS_B = NPAD // NW
SUB_ROWS = NPAD // 16


def _sc_build_b(src, dst, ew):
    mesh = plsc.VectorSubcoreMesh(core_axis_name="c", subcore_axis_name="s")
    CBLK = 2000
    NBLK = E // CBLK

    @functools.partial(
        pl.kernel,
        out_type=jax.ShapeDtypeStruct((NPAD * K,), jnp.float32),
        mesh=mesh,
        compiler_params=pltpu.CompilerParams(needs_layout_passes=False),
        scratch_types=[
            pltpu.VMEM((ROWS_B * K,), jnp.float32),
            pltpu.VMEM((CBLK,), jnp.int32),
            pltpu.VMEM((CBLK,), jnp.int32),
            pltpu.VMEM((CBLK,), jnp.float32),
            pltpu.VMEM((CBLK,), jnp.int32),
            pltpu.VMEM((CBLK,), jnp.int32),
            pltpu.VMEM((CBLK,), jnp.float32),
            pltpu.SemaphoreType.DMA,
            pltpu.SemaphoreType.DMA,
            pltpu.SemaphoreType.DMA,
            pltpu.SemaphoreType.DMA,
            pltpu.SemaphoreType.DMA,
            pltpu.SemaphoreType.DMA,
        ],
    )
    def build(src_hbm, dst_hbm, ew_hbm, b_hbm, bt,
              sb0, db0, wb0, sb1, db1, wb1, s0, s1, s2, s3, s4, s5):
        wid = lax.axis_index("s") * 2 + lax.axis_index("c")
        base = wid * ROWS_B
        zero = jnp.zeros((16,), jnp.float32)

        def zrow(i, _):
            bt[pl.ds(i * 16, 16)] = zero
            return 0
        lax.fori_loop(0, ROWS_B * K // 16, zrow, 0, unroll=8)

        bufs = ((sb0, db0, wb0, s0, s1, s2), (sb1, db1, wb1, s3, s4, s5))

        def fire(b, p):
            sbb, dbb, wbb, ss, sd, sw = bufs[p]
            e0 = b * CBLK
            pltpu.async_copy(src_hbm.at[pl.ds(e0, CBLK)], sbb, ss)
            pltpu.async_copy(dst_hbm.at[pl.ds(e0, CBLK)], dbb, sd)
            pltpu.async_copy(ew_hbm.at[pl.ds(e0, CBLK)], wbb, sw)

        def scan(b, p):
            sbb, dbb, wbb, ss, sd, sw = bufs[p]
            e0 = b * CBLK
            pltpu.make_async_copy(src_hbm.at[pl.ds(e0, CBLK)], sbb, ss).wait()
            pltpu.make_async_copy(dst_hbm.at[pl.ds(e0, CBLK)], dbb, sd).wait()
            pltpu.make_async_copy(ew_hbm.at[pl.ds(e0, CBLK)], wbb, sw).wait()

            def grp(g, _):
                d = dbb[pl.ds(g * 16, 16)]
                s = sbb[pl.ds(g * 16, 16)]
                w = wbb[pl.ds(g * 16, 16)]
                hs = lax.bitwise_and(s, K - 1)
                r = d - base
                m = (d >= base) & (d < base + ROWS_B)
                idx = jnp.where(m, lax.shift_left(r, 8) + hs, 0)
                plsc.addupdate_scatter(bt, [idx], w, mask=m)
                return 0
            lax.fori_loop(0, CBLK // 16, grp, 0, unroll=5)

        fire(0, 0)

        def pairloop(i, _):
            c = 2 * i
            fire(c + 1, 1)
            scan(c, 0)

            @pl.when(i < NBLK // 2 - 1)
            def _():
                fire(c + 2, 0)
            scan(c + 1, 1)
            return 0
        lax.fori_loop(0, NBLK // 2, pairloop, 0)
        pltpu.sync_copy(bt, b_hbm.at[pl.ds(base * K, ROWS_B * K)])

    return build(src, dst, ew)


def _sc_propagate(table, src, dst, ew):
    mesh = plsc.VectorSubcoreMesh(core_axis_name="c", subcore_axis_name="s")
    CG = 80
    NCH = EPW // CG
    NTRI = (NCH - 2) // 3

    CB = 2000
    CPB = CB // CG

    @functools.partial(
        pl.kernel,
        out_type=jax.ShapeDtypeStruct((2, NPAD, H), jnp.float32),
        mesh=mesh,
        compiler_params=pltpu.CompilerParams(needs_layout_passes=False),
        scratch_types=[
            pltpu.VMEM_SHARED((NPAD, H), jnp.float32),
            pltpu.VMEM((CG, H), jnp.float32),
            pltpu.VMEM((CG, H), jnp.float32),
            pltpu.VMEM((CB,), jnp.int32),
            pltpu.VMEM((CB,), jnp.int32),
            pltpu.VMEM((CB,), jnp.float32),
            pltpu.VMEM((CG, H), jnp.float32),
            pltpu.VMEM((CG,), jnp.int32),
            pltpu.VMEM((CG,), jnp.int32),
            pltpu.VMEM((CG,), jnp.int32),
            pltpu.VMEM((CG,), jnp.int32),
            pltpu.VMEM((CG,), jnp.int32),
            pltpu.VMEM((CG,), jnp.int32),
            pltpu.VMEM((CG,), jnp.float32),
            pltpu.VMEM((CG,), jnp.float32),
            pltpu.VMEM((CG,), jnp.float32),
            pltpu.SemaphoreType.DMA,
            pltpu.SemaphoreType.DMA,
            pltpu.SemaphoreType.DMA,
            pltpu.SemaphoreType.DMA,
            pltpu.SemaphoreType.DMA,
            pltpu.SemaphoreType.DMA,
        ],
    )
    def prop(tab_hbm, src_hbm, dst_hbm, ew_hbm, out_hbm,
             acc, rows0, rows1, sb, db, wb, rows2,
             sc0, sc1, sc2, dc0, dc1, dc2, wc0, wc1, wc2,
             gsem0, gsem1, gsem2, ssem0, ssem1, ssem2):
        cid = lax.axis_index("c")
        sid = lax.axis_index("s")
        wid = sid * 2 + cid
        zero = jnp.zeros((16,), jnp.float32)

        def zrow(i, _):
            for j in range(H // 16):
                rows0[i, pl.ds(j * 16, 16)] = zero
            return 0
        lax.fori_loop(0, CG, zrow, 0)

        def zstripe(i, _):
            pltpu.sync_copy(rows0, acc.at[pl.ds(sid * SUB_ROWS + i * CG, CG)])
            return 0
        lax.fori_loop(0, SUB_ROWS // CG, zstripe, 0)
        plsc.subcore_barrier()

        def stage_fire(ci, scb, dcb, wcb, rowsb, gsem):
            bi = lax.div(ci, CPB)
            off = lax.rem(ci, CPB) * CG

            @pl.when(off == 0)
            def _():
                e0 = wid * EPW + bi * CB
                pltpu.sync_copy(src_hbm.at[pl.ds(e0, CB)], sb)
                pltpu.sync_copy(dst_hbm.at[pl.ds(e0, CB)], db)
                pltpu.sync_copy(ew_hbm.at[pl.ds(e0, CB)], wb)
            for q in range(CG // 16):
                scb[pl.ds(q * 16, 16)] = sb[pl.ds(off + q * 16, 16)]
                dcb[pl.ds(q * 16, 16)] = db[pl.ds(off + q * 16, 16)]
                wcb[pl.ds(q * 16, 16)] = wb[pl.ds(off + q * 16, 16)]
            pltpu.async_copy(tab_hbm.at[scb], rowsb, gsem)

        def process(scb, dcb, wcb, rowsb, gsem, ssem):
            pltpu.make_async_copy(tab_hbm.at[scb], rowsb, gsem).wait()

            @functools.partial(plsc.parallel_loop, 0, CG, unroll=8)
            def edge(i):
                ewv = plsc.load_gather(wcb, [jnp.zeros((16,), jnp.int32) + i])
                for j in range(H // 16):
                    rowsb[i, pl.ds(j * 16, 16)] = (
                        rowsb[i, pl.ds(j * 16, 16)] * ewv)
            pltpu.async_copy(rowsb, acc.at[dcb], ssem, add=True)

        def wait_scatter(rowsb, dcb, ssem):
            pltpu.make_async_copy(rowsb, acc.at[dcb], ssem).wait()

        bufs = ((sc0, dc0, wc0, rows0, gsem0, ssem0),
                (sc1, dc1, wc1, rows1, gsem1, ssem1),
                (sc2, dc2, wc2, rows2, gsem2, ssem2))

        stage_fire(0, *bufs[0][:5])
        stage_fire(1, *bufs[1][:5])

        def triple(i, _):
            for k in range(3):
                c = 3 * i + k
                scb, dcb, wcb, rowsb, gsem, ssem = bufs[k]
                nscb, ndcb, nwcb, nrowsb, ngsem, nssem = bufs[(k + 2) % 3]
                process(scb, dcb, wcb, rowsb, gsem, ssem)

                @pl.when(c >= 1)
                def _():
                    wait_scatter(nrowsb, ndcb, nssem)
                stage_fire(c + 2, nscb, ndcb, nwcb, nrowsb, ngsem)
            return 0
        lax.fori_loop(0, NTRI, triple, 0)
        process(*bufs[0])
        process(*bufs[1])
        wait_scatter(rows0, dc0, ssem0)
        wait_scatter(rows1, dc1, ssem1)
        wait_scatter(rows2, dc2, ssem2)

        plsc.subcore_barrier()
        pltpu.sync_copy(acc.at[pl.ds(sid * SUB_ROWS, SUB_ROWS)],
                        out_hbm.at[cid, pl.ds(sid * SUB_ROWS, SUB_ROWS)])

    return prop(table, src, dst, ew)


_BLK = 1024
_GRID = NPAD // _BLK


def _dot(a, b):
    return jnp.dot(a, b, preferred_element_type=jnp.float32)


def _elu(x):
    return jnp.where(x > 0, x, jnp.exp(jnp.minimum(x, 0.0)) - 1.0)


def _tc_reduce_b(B):
    def body(b_ref, a_ref, dinv_ref, dcinv_ref):
        i = pl.program_id(0)
        blk = b_ref[...]
        part = (blk[0:256] + blk[256:512] + blk[512:768] + blk[768:1024])

        @pl.when(i == 0)
        def _():
            a_ref[...] = jnp.zeros_like(a_ref)
        a_ref[...] += part
        dinv_ref[...] = 1.0 / (jnp.sum(blk, axis=1, keepdims=True) + 1.0)

        @pl.when(i == _GRID - 1)
        def _():
            dcinv_ref[...] = 1.0 / (
                jnp.sum(a_ref[...], axis=1, keepdims=True) + 1.0)

    return pl.pallas_call(
        body,
        grid=(_GRID,),
        in_specs=[pl.BlockSpec((_BLK, K), lambda i: (i, 0))],
        out_specs=[
            pl.BlockSpec((K, K), lambda i: (0, 0)),
            pl.BlockSpec((_BLK, 1), lambda i: (i, 0)),
            pl.BlockSpec((K, 1), lambda i: (0, 0)),
        ],
        out_shape=[
            jax.ShapeDtypeStruct((K, K), jnp.float32),
            jax.ShapeDtypeStruct((NPAD, 1), jnp.float32),
            jax.ShapeDtypeStruct((K, 1), jnp.float32),
        ],
    )(B)


def _tc_embed(xcat, W1, b1, W2, b2):
    def body(x_ref, w1_ref, b1_ref, w2_ref, b2_ref, o_ref):
        h = _elu(_dot(x_ref[...], w1_ref[...]) + b1_ref[...])
        o_ref[...] = _elu(_dot(h, w2_ref[...]) + b2_ref[...])

    return pl.pallas_call(
        body,
        grid=(_GRID,),
        in_specs=[
            pl.BlockSpec((_BLK, 8), lambda i: (i, 0)),
            pl.BlockSpec((8, H), lambda i: (0, 0)),
            pl.BlockSpec((1, H), lambda i: (0, 0)),
            pl.BlockSpec((H, H), lambda i: (0, 0)),
            pl.BlockSpec((1, H), lambda i: (0, 0)),
        ],
        out_specs=pl.BlockSpec((_BLK, H), lambda i: (i, 0)),
        out_shape=jax.ShapeDtypeStruct((NPAD, H), jnp.float32),
    )(xcat, W1, b1, W2, b2)


def _tc_gcn_fin(p, h, deginv, W, b):
    def body(p_ref, h_ref, d_ref, w_ref, b_ref, o_ref):
        z = (p_ref[0] + p_ref[1] + h_ref[...]) * d_ref[...]
        o_ref[...] = _elu(_dot(z, w_ref[...]) + b_ref[...])

    return pl.pallas_call(
        body,
        grid=(_GRID,),
        in_specs=[
            pl.BlockSpec((2, _BLK, H), lambda i: (0, i, 0)),
            pl.BlockSpec((_BLK, H), lambda i: (i, 0)),
            pl.BlockSpec((_BLK, 1), lambda i: (i, 0)),
            pl.BlockSpec((H, H), lambda i: (0, 0)),
            pl.BlockSpec((1, H), lambda i: (0, 0)),
        ],
        out_specs=pl.BlockSpec((_BLK, H), lambda i: (i, 0)),
        out_shape=jax.ShapeDtypeStruct((NPAD, H), jnp.float32),
    )(p, h, deginv, W, b)


def _tc_gcn_pool(p, h, deginv, W, b, cntinv):
    def body(p_ref, h_ref, d_ref, w_ref, b_ref, c_ref, pool_ref):
        i = pl.program_id(0)
        z = (p_ref[0] + p_ref[1] + h_ref[...]) * d_ref[...]
        hf = _elu(_dot(z, w_ref[...]) + b_ref[...])
        gid = i * _BLK + lax.broadcasted_iota(jnp.int32, (_BLK, 1), 0)
        hf = jnp.where(gid < N, hf, 0.0)
        part = hf[0:256] + hf[256:512] + hf[512:768] + hf[768:1024]

        @pl.when(i == 0)
        def _():
            pool_ref[...] = jnp.zeros_like(pool_ref)
        pool_ref[...] += part

        @pl.when(i == _GRID - 1)
        def _():
            pool_ref[...] *= c_ref[...]

    return pl.pallas_call(
        body,
        grid=(_GRID,),
        in_specs=[
            pl.BlockSpec((2, _BLK, H), lambda i: (0, i, 0)),
            pl.BlockSpec((_BLK, H), lambda i: (i, 0)),
            pl.BlockSpec((_BLK, 1), lambda i: (i, 0)),
            pl.BlockSpec((H, H), lambda i: (0, 0)),
            pl.BlockSpec((1, H), lambda i: (0, 0)),
            pl.BlockSpec((K, 1), lambda i: (0, 0)),
        ],
        out_specs=pl.BlockSpec((K, H), lambda i: (0, 0)),
        out_shape=jax.ShapeDtypeStruct((K, H), jnp.float32),
    )(p, h, deginv, W, b, cntinv)


def _tc_lstm(A, degcinv, xin, hprev, cprev, Wx, Wh, bcat, wci, wcf, wco):
    def body(a_ref, dc_ref, x_ref, h_ref, c_ref, wx_ref, wh_ref, b_ref,
             wci_ref, wcf_ref, wco_ref, hn_ref, cn_ref):
        a = a_ref[...]
        dinv = dc_ref[...]
        x = x_ref[...]
        hp = h_ref[...]
        c = c_ref[...]
        xp = (_dot(a, x) + x) * dinv
        hpp = (_dot(a, hp) + hp) * dinv
        g = _dot(xp, wx_ref[...]) + _dot(hpp, wh_ref[...]) + b_ref[...]
        gi, gf, gc, go = g[:, 0:128], g[:, 128:256], g[:, 256:384], g[:, 384:512]
        i = jax.nn.sigmoid(gi + wci_ref[...] * c)
        f = jax.nn.sigmoid(gf + wcf_ref[...] * c)
        cn = f * c + i * jnp.tanh(gc)
        o = jax.nn.sigmoid(go + wco_ref[...] * cn)
        hn_ref[...] = o * jnp.tanh(cn)
        cn_ref[...] = cn

    return pl.pallas_call(
        body,
        out_shape=[
            jax.ShapeDtypeStruct((K, H), jnp.float32),
            jax.ShapeDtypeStruct((K, H), jnp.float32),
        ],
    )(A, degcinv, xin, hprev, cprev, Wx, Wh, bcat, wci, wcf, wco)


def _tc_dec_init(A, degcinv, y, Winit, binit):
    def body(a_ref, dc_ref, y_ref, w_ref, b_ref, o_ref):
        y = y_ref[...]
        z = (_dot(a_ref[...], y) + y) * dc_ref[...]
        o_ref[...] = _elu(_dot(z, w_ref[...]) + b_ref[...])

    return pl.pallas_call(
        body,
        out_shape=jax.ShapeDtypeStruct((K, 2 * H), jnp.float32),
    )(A, degcinv, y, Winit, binit)


def _tc_dec_l0(B, dh, deginv, W, b):
    def body(b_ref, dh_ref, d_ref, w_ref, bb_ref, o_ref):
        dh = dh_ref[...]
        agg = _dot(b_ref[...], dh)
        u = jnp.concatenate([dh, dh, dh, dh], axis=0)
        z = (agg + u) * d_ref[...]
        o_ref[...] = _elu(_dot(z, w_ref[...]) + bb_ref[...])

    return pl.pallas_call(
        body,
        grid=(_GRID,),
        in_specs=[
            pl.BlockSpec((_BLK, K), lambda i: (i, 0)),
            pl.BlockSpec((K, H), lambda i: (0, 0)),
            pl.BlockSpec((_BLK, 1), lambda i: (i, 0)),
            pl.BlockSpec((H, H), lambda i: (0, 0)),
            pl.BlockSpec((1, H), lambda i: (0, 0)),
        ],
        out_specs=pl.BlockSpec((_BLK, H), lambda i: (i, 0)),
        out_shape=jax.ShapeDtypeStruct((NPAD, H), jnp.float32),
    )(B, dh, deginv, W, b)


def _tc_head(p, u, deginv, W, b, Wo1, bo1, Wo2, bo2):
    def body(p_ref, u_ref, d_ref, w_ref, b_ref, w1_ref, b1_ref, w2_ref,
             b2_ref, o_ref):
        z = (p_ref[0] + p_ref[1] + u_ref[...]) * d_ref[...]
        z = _elu(_dot(z, w_ref[...]) + b_ref[...])
        z = _elu(_dot(z, w1_ref[...]) + b1_ref[...])
        o_ref[...] = _dot(z, w2_ref[...]) + b2_ref[...]

    return pl.pallas_call(
        body,
        grid=(_GRID,),
        in_specs=[
            pl.BlockSpec((2, _BLK, H), lambda i: (0, i, 0)),
            pl.BlockSpec((_BLK, H), lambda i: (i, 0)),
            pl.BlockSpec((_BLK, 1), lambda i: (i, 0)),
            pl.BlockSpec((H, H), lambda i: (0, 0)),
            pl.BlockSpec((1, H), lambda i: (0, 0)),
            pl.BlockSpec((H, H), lambda i: (0, 0)),
            pl.BlockSpec((1, H), lambda i: (0, 0)),
            pl.BlockSpec((H, 8), lambda i: (0, 0)),
            pl.BlockSpec((1, 8), lambda i: (0, 0)),
        ],
        out_specs=pl.BlockSpec((_BLK, 8), lambda i: (i, 0)),
        out_shape=jax.ShapeDtypeStruct((NPAD, 8), jnp.float32),
    )(p, u, deginv, W, b, Wo1, bo1, Wo2, bo2)


def kernel(x, seq_len, edge_index, edge_attr, pos, params):
    p = params
    src = edge_index[0].astype(jnp.int32)
    dst = edge_index[1].astype(jnp.int32)
    ew = edge_attr

    B = _sc_build_b(src, dst, ew).reshape(NPAD, K)
    A, deginv, degcinv = _tc_reduce_b(B)

    cnt = np.where(np.arange(K) < (N % K), N // K + 1, N // K)
    cntinv = jnp.asarray(1.0 / cnt, jnp.float32).reshape(K, 1)

    W1p = jnp.pad(p["We1"], ((0, 3), (0, 0)))
    b1 = p["be1"].reshape(1, H)
    W2 = p["We2"]
    b2 = p["be2"].reshape(1, H)

    def gcat(pre, kind):
        return jnp.concatenate(
            [p[f"{pre}_W{kind}_{g}"] for g in ("i", "f", "c", "o")], axis=1)

    Wx_e, Wh_e = gcat("enc", "x"), gcat("enc", "h")
    b_e = jnp.concatenate(
        [p[f"enc_b_{g}"] for g in ("i", "f", "c", "o")]).reshape(1, 4 * H)
    Wx_d, Wh_d = gcat("dec", "x"), gcat("dec", "h")
    b_d = jnp.concatenate(
        [p[f"dec_b_{g}"] for g in ("i", "f", "c", "o")]).reshape(1, 4 * H)
    wci_e = p["enc_wci"].reshape(1, H)
    wcf_e = p["enc_wcf"].reshape(1, H)
    wco_e = p["enc_wco"].reshape(1, H)
    wci_d = p["dec_wci"].reshape(1, H)
    wcf_d = p["dec_wcf"].reshape(1, H)
    wco_d = p["dec_wco"].reshape(1, H)
    binit = p["binit"].reshape(1, 2 * H)
    bo1 = p["bo1"].reshape(1, H)
    Wo2p = jnp.pad(p["Wo2"], ((0, 0), (0, 8 - COUT)))
    bo2p = jnp.pad(p["bo2"], (0, 8 - COUT)).reshape(1, 8)

    T = x.shape[0]
    pos_b = jnp.broadcast_to(pos[None], (T, N, 2))
    xcat = jnp.concatenate(
        [x, pos_b, jnp.zeros((T, N, 3), jnp.float32)], axis=2)
    xcat = jnp.pad(xcat, ((0, 0), (0, NPAD - N), (0, 0)))

    h_t = jnp.zeros((K, H), jnp.float32)
    c_t = jnp.zeros((K, H), jnp.float32)
    for t in range(T):
        hf = _tc_embed(xcat[t], W1p, b1, W2, b2)
        pr = _sc_propagate(hf, src, dst, ew)
        hf = _tc_gcn_fin(pr, hf, deginv, p["Wenc_mp0"],
                         p["benc_mp0"].reshape(1, H))
        pr = _sc_propagate(hf, src, dst, ew)
        pooled = _tc_gcn_pool(pr, hf, deginv, p["Wenc_mp1"],
                              p["benc_mp1"].reshape(1, H), cntinv)
        h_t, c_t = _tc_lstm(A, degcinv, pooled, h_t, c_t,
                            Wx_e, Wh_e, b_e, wci_e, wcf_e, wco_e)

    y = jnp.concatenate([h_t, c_t], axis=1)
    comb = _tc_dec_init(A, degcinv, y, p["Winit"], binit)
    dh0, dc0 = comb[:, :H], comb[:, H:]

    def dec_body(t, carry):
        dh, dc, out = carry
        dh, dc = _tc_lstm(A, degcinv, h_t, dh, dc,
                          Wx_d, Wh_d, b_d, wci_d, wcf_d, wco_d)
        u = _tc_dec_l0(B, dh, deginv, p["Wdec_mp0"],
                       p["bdec_mp0"].reshape(1, H))
        pr = _sc_propagate(u, src, dst, ew)
        o = _tc_head(pr, u, deginv, p["Wdec_mp1"],
                     p["bdec_mp1"].reshape(1, H),
                     p["Wo1"], bo1, Wo2p, bo2p)
        out = lax.dynamic_update_index_in_dim(out, o, t, axis=0)
        return dh, dc, out

    out0 = jnp.zeros((4, NPAD, 8), jnp.float32)
    _, _, out = lax.fori_loop(0, seq_len, dec_body, (dh0, dc0, out0))
    return out[:, :N, :COUT]
